# baseline (speedup 1.0000x reference)
import jax
import jax.numpy as jnp
from jax import lax
from jax.experimental import pallas as pl
from jax.experimental.pallas import tpu as pltpu
from jax.experimental.pallas import tpu_sc as plsc

T = 32
E = 64
D = 1024
L = 16
NCH = E // L


def _gate_body(x_ref, wg_ref, bg_ref, o_ref):
    o_ref[...] = (
        jnp.dot(x_ref[...], wg_ref[...], preferred_element_type=jnp.float32)
        + bg_ref[...]
    )


def _gate(x, Wg, bg):
    return pl.pallas_call(
        _gate_body,
        out_shape=jax.ShapeDtypeStruct((T, E), jnp.float32),
    )(x, Wg, bg.reshape(1, E))


def _route_body(logits_hbm, ids_hbm, ws_hbm, nact_hbm, lg, wq, ids_v, ws_v, nact_v):
    c = lax.axis_index("c")
    s = lax.axis_index("s")

    @pl.when(jnp.logical_and(c == 0, s == 0))
    def _():
        pltpu.sync_copy(logits_hbm, lg)
        iota = lax.iota(jnp.int32, L)
        gidx = [iota + L * k for k in range(NCH)]
        NEG = jnp.float32(-1e30)
        BIG = jnp.int32(9999)

        def token_body(t, acc):
            lk = [lg[t, pl.ds(L * k, L)] for k in range(NCH)]
            m = jnp.max(lk[0])
            for k in range(1, NCH):
                m = jnp.maximum(m, jnp.max(lk[k]))
            mv = jnp.full((L,), m, jnp.float32)
            ek = [jnp.exp(lk[k] - mv) for k in range(NCH)]
            Z = jnp.sum(ek[0])
            for k in range(1, NCH):
                Z = Z + jnp.sum(ek[k])
            Zv = jnp.full((L,), Z, jnp.float32)
            idx1 = jnp.min(jnp.where(lk[0] == mv, gidx[0], jnp.full((L,), BIG)))
            for k in range(1, NCH):
                idx1 = jnp.minimum(
                    idx1,
                    jnp.min(jnp.where(lk[k] == mv, gidx[k], jnp.full((L,), BIG))),
                )
            i1v = jnp.full((L,), idx1, jnp.int32)
            l2k = [
                jnp.where(gidx[k] == i1v, jnp.full((L,), NEG), lk[k])
                for k in range(NCH)
            ]
            m2 = jnp.max(l2k[0])
            for k in range(1, NCH):
                m2 = jnp.maximum(m2, jnp.max(l2k[k]))
            m2v = jnp.full((L,), m2, jnp.float32)
            idx2 = jnp.min(jnp.where(l2k[0] == m2v, gidx[0], jnp.full((L,), BIG)))
            for k in range(1, NCH):
                idx2 = jnp.minimum(
                    idx2,
                    jnp.min(jnp.where(l2k[k] == m2v, gidx[k], jnp.full((L,), BIG))),
                )
            i2v = jnp.full((L,), idx2, jnp.int32)
            new_acc = []
            for k in range(NCH):
                selk = jnp.logical_or(gidx[k] == i1v, gidx[k] == i2v)
                wk = jnp.where(selk, ek[k] / Zv, jnp.zeros((L,), jnp.float32))
                wq[t, pl.ds(L * k, L)] = wk
                new_acc.append(jnp.maximum(acc[k], wk))
            return tuple(new_acc)

        zero = jnp.zeros((L,), jnp.float32)
        acc = lax.fori_loop(0, T, token_body, (zero,) * NCH)

        off = jnp.int32(0)
        for k in range(NCH):
            selk = acc[k] > 0.0
            sel_i = jnp.where(selk, jnp.ones((L,), jnp.int32), jnp.zeros((L,), jnp.int32))
            cs = plsc.cumsum(sel_i) + jnp.full((L,), off, jnp.int32)
            pos = jnp.maximum(cs - 1, jnp.zeros((L,), jnp.int32))
            plsc.store_scatter(ids_v, [pos], gidx[k], mask=selk)
            off = off + jnp.sum(sel_i)
        n = off
        lastv = plsc.load_gather(ids_v, [jnp.full((L,), n - 1, jnp.int32)])
        for k in range(NCH):
            cur = ids_v[pl.ds(L * k, L)]
            nv = jnp.full((L,), n, jnp.int32)
            ids_v[pl.ds(L * k, L)] = jnp.where(gidx[k] >= nv, lastv, cur)
        nact_v[...] = jnp.full((L,), n, jnp.int32)

        def ws_body(i, carry):
            ev = plsc.load_gather(ids_v, [jnp.full((L,), i, jnp.int32)])
            scale = jnp.where(i < n, jnp.float32(1.0), jnp.float32(0.0))
            sv = jnp.full((L,), scale, jnp.float32)
            for h in range(T // L):
                tv = iota + L * h
                vals = plsc.load_gather(wq, [tv, ev]) * sv
                ws_v[i, pl.ds(L * h, L)] = vals
            return carry

        lax.fori_loop(0, E, ws_body, jnp.int32(0))

        pltpu.sync_copy(ids_v, ids_hbm)
        pltpu.sync_copy(ws_v, ws_hbm)
        pltpu.sync_copy(nact_v, nact_hbm)


def _route(gate_logits):
    mesh = plsc.VectorSubcoreMesh(core_axis_name="c", subcore_axis_name="s")
    return pl.kernel(
        _route_body,
        compiler_params=pltpu.CompilerParams(needs_layout_passes=False),
        out_type=(
            jax.ShapeDtypeStruct((E,), jnp.int32),
            jax.ShapeDtypeStruct((E, T), jnp.float32),
            jax.ShapeDtypeStruct((L,), jnp.int32),
        ),
        mesh=mesh,
        scratch_types=(
            pltpu.VMEM((T, E), jnp.float32),
            pltpu.VMEM((T, E), jnp.float32),
            pltpu.VMEM((E,), jnp.int32),
            pltpu.VMEM((E, T), jnp.float32),
            pltpu.VMEM((L,), jnp.int32),
        ),
    )(gate_logits)


_KSPLIT = 4
_KS = D // _KSPLIT


def _moe_body(ids_ref, n_ref, x_ref, *rest):
    we_refs = rest[:_KSPLIT]
    be_ref, ws_ref, o_ref = rest[_KSPLIT:]
    i = pl.program_id(0)

    @pl.when(i == 0)
    def _init():
        o_ref[...] = jnp.zeros_like(o_ref)

    @pl.when(i < n_ref[0])
    def _step():
        y = jnp.dot(
            x_ref[:, pl.ds(0, _KS)], we_refs[0][0],
            preferred_element_type=jnp.float32,
        )
        for p in range(1, _KSPLIT):
            y += jnp.dot(
                x_ref[:, pl.ds(p * _KS, _KS)], we_refs[p][0],
                preferred_element_type=jnp.float32,
            )
        w = ws_ref[0, 0, :]
        o_ref[...] += (y + be_ref[0]) * w[:, None]


def _moe(ids, nact, x, We, be, ws):
    we_spec = [
        pl.BlockSpec((1, _KS, D), lambda i, ids, n, p=p: (ids[i], p, 0))
        for p in range(_KSPLIT)
    ]
    grid_spec = pltpu.PrefetchScalarGridSpec(
        num_scalar_prefetch=2,
        grid=(E,),
        in_specs=[
            pl.BlockSpec((T, D), lambda i, ids, n: (0, 0)),
            *we_spec,
            pl.BlockSpec((1, 1, D), lambda i, ids, n: (ids[i], 0, 0)),
            pl.BlockSpec((1, 1, T), lambda i, ids, n: (i, 0, 0)),
        ],
        out_specs=pl.BlockSpec((T, D), lambda i, ids, n: (0, 0)),
    )
    return pl.pallas_call(
        _moe_body,
        grid_spec=grid_spec,
        out_shape=jax.ShapeDtypeStruct((T, D), jnp.float32),
        compiler_params=pltpu.CompilerParams(
            dimension_semantics=("arbitrary",)
        ),
    )(ids, nact, x, *([We] * _KSPLIT), be.reshape(E, 1, D), ws.reshape(E, 1, T))


def kernel(inputs, Wg, bg, We, be, k):
    del k
    gl = inputs @ Wg + bg
    m1 = jnp.max(gl, axis=1, keepdims=True)
    e = jnp.exp(gl - m1)
    W = e / jnp.sum(e, axis=1, keepdims=True)
    i1 = jnp.argmax(gl, axis=1)
    gl2 = gl.at[jnp.arange(T), i1].set(-1e30)
    i2 = jnp.argmax(gl2, axis=1)
    mask = (jax.nn.one_hot(i1, E) + jax.nn.one_hot(i2, E)) > 0
    Wm = jnp.where(mask, W, 0.0)
    ids = jnp.arange(E, dtype=jnp.int32)
    nact = jnp.full((16,), E, jnp.int32)
    ws = Wm.T
    return _moe(ids, nact, inputs, We, be, ws)


# speedup vs baseline: 1.1599x; 1.1599x over previous
import jax
import jax.numpy as jnp
from jax import lax
from jax.experimental import pallas as pl
from jax.experimental.pallas import tpu as pltpu
from jax.experimental.pallas import tpu_sc as plsc

T = 32
E = 64
D = 1024
L = 16
NCH = E // L


def _gate_body(x_ref, wg_ref, bg_ref, o_ref):
    o_ref[...] = (
        jnp.dot(x_ref[...], wg_ref[...], preferred_element_type=jnp.float32)
        + bg_ref[...]
    )


def _gate(x, Wg, bg):
    return pl.pallas_call(
        _gate_body,
        out_shape=jax.ShapeDtypeStruct((T, E), jnp.float32),
    )(x, Wg, bg.reshape(1, E))


def _route_body(logits_hbm, ids_hbm, ws_hbm, nact_hbm, lg, wq, ids_v, ws_v, nact_v):
    c = lax.axis_index("c")
    s = lax.axis_index("s")

    @pl.when(jnp.logical_and(c == 0, s == 0))
    def _():
        pltpu.sync_copy(logits_hbm, lg)
        iota = lax.iota(jnp.int32, L)
        gidx = [iota + L * k for k in range(NCH)]
        NEG = jnp.float32(-1e30)
        BIG = jnp.int32(9999)

        def token_body(t, acc):
            lk = [lg[t, pl.ds(L * k, L)] for k in range(NCH)]
            m = jnp.max(lk[0])
            for k in range(1, NCH):
                m = jnp.maximum(m, jnp.max(lk[k]))
            mv = jnp.full((L,), m, jnp.float32)
            ek = [jnp.exp(lk[k] - mv) for k in range(NCH)]
            Z = jnp.sum(ek[0])
            for k in range(1, NCH):
                Z = Z + jnp.sum(ek[k])
            Zv = jnp.full((L,), Z, jnp.float32)
            idx1 = jnp.min(jnp.where(lk[0] == mv, gidx[0], jnp.full((L,), BIG)))
            for k in range(1, NCH):
                idx1 = jnp.minimum(
                    idx1,
                    jnp.min(jnp.where(lk[k] == mv, gidx[k], jnp.full((L,), BIG))),
                )
            i1v = jnp.full((L,), idx1, jnp.int32)
            l2k = [
                jnp.where(gidx[k] == i1v, jnp.full((L,), NEG), lk[k])
                for k in range(NCH)
            ]
            m2 = jnp.max(l2k[0])
            for k in range(1, NCH):
                m2 = jnp.maximum(m2, jnp.max(l2k[k]))
            m2v = jnp.full((L,), m2, jnp.float32)
            idx2 = jnp.min(jnp.where(l2k[0] == m2v, gidx[0], jnp.full((L,), BIG)))
            for k in range(1, NCH):
                idx2 = jnp.minimum(
                    idx2,
                    jnp.min(jnp.where(l2k[k] == m2v, gidx[k], jnp.full((L,), BIG))),
                )
            i2v = jnp.full((L,), idx2, jnp.int32)
            new_acc = []
            for k in range(NCH):
                selk = jnp.logical_or(gidx[k] == i1v, gidx[k] == i2v)
                wk = jnp.where(selk, ek[k] / Zv, jnp.zeros((L,), jnp.float32))
                wq[t, pl.ds(L * k, L)] = wk
                new_acc.append(jnp.maximum(acc[k], wk))
            return tuple(new_acc)

        zero = jnp.zeros((L,), jnp.float32)
        acc = lax.fori_loop(0, T, token_body, (zero,) * NCH)

        off = jnp.int32(0)
        for k in range(NCH):
            selk = acc[k] > 0.0
            sel_i = jnp.where(selk, jnp.ones((L,), jnp.int32), jnp.zeros((L,), jnp.int32))
            cs = plsc.cumsum(sel_i) + jnp.full((L,), off, jnp.int32)
            pos = jnp.maximum(cs - 1, jnp.zeros((L,), jnp.int32))
            plsc.store_scatter(ids_v, [pos], gidx[k], mask=selk)
            off = off + jnp.sum(sel_i)
        n = off
        lastv = plsc.load_gather(ids_v, [jnp.full((L,), n - 1, jnp.int32)])
        for k in range(NCH):
            cur = ids_v[pl.ds(L * k, L)]
            nv = jnp.full((L,), n, jnp.int32)
            ids_v[pl.ds(L * k, L)] = jnp.where(gidx[k] >= nv, lastv, cur)
        nact_v[...] = jnp.full((L,), n, jnp.int32)

        def ws_body(i, carry):
            ev = plsc.load_gather(ids_v, [jnp.full((L,), i, jnp.int32)])
            scale = jnp.where(i < n, jnp.float32(1.0), jnp.float32(0.0))
            sv = jnp.full((L,), scale, jnp.float32)
            for h in range(T // L):
                tv = iota + L * h
                vals = plsc.load_gather(wq, [tv, ev]) * sv
                ws_v[i, pl.ds(L * h, L)] = vals
            return carry

        lax.fori_loop(0, E, ws_body, jnp.int32(0))

        pltpu.sync_copy(ids_v, ids_hbm)
        pltpu.sync_copy(ws_v, ws_hbm)
        pltpu.sync_copy(nact_v, nact_hbm)


def _route(gate_logits):
    mesh = plsc.VectorSubcoreMesh(core_axis_name="c", subcore_axis_name="s")
    return pl.kernel(
        _route_body,
        compiler_params=pltpu.CompilerParams(needs_layout_passes=False),
        out_type=(
            jax.ShapeDtypeStruct((E,), jnp.int32),
            jax.ShapeDtypeStruct((E, T), jnp.float32),
            jax.ShapeDtypeStruct((L,), jnp.int32),
        ),
        mesh=mesh,
        scratch_types=(
            pltpu.VMEM((T, E), jnp.float32),
            pltpu.VMEM((T, E), jnp.float32),
            pltpu.VMEM((E,), jnp.int32),
            pltpu.VMEM((E, T), jnp.float32),
            pltpu.VMEM((L,), jnp.int32),
        ),
    )(gate_logits)


_KSPLIT = 4
_KS = D // _KSPLIT


def _moe_body(ids_ref, x_ref, *rest):
    we_refs = rest[:_KSPLIT]
    be_ref, ws_ref, o_ref = rest[_KSPLIT:]
    i = pl.program_id(0)

    @pl.when(i == 0)
    def _init():
        o_ref[...] = jnp.zeros_like(o_ref)

    y = jnp.dot(
        x_ref[:, pl.ds(0, _KS)], we_refs[0][0],
        preferred_element_type=jnp.float32,
    )
    for p in range(1, _KSPLIT):
        y += jnp.dot(
            x_ref[:, pl.ds(p * _KS, _KS)], we_refs[p][0],
            preferred_element_type=jnp.float32,
        )
    w = ws_ref[0, 0, :]
    o_ref[...] += (y + be_ref[0]) * w[:, None]


def _moe(ids, nact, x, We, be, ws):
    n = nact[0]
    we_spec = [
        pl.BlockSpec((1, _KS, D), lambda i, ids, p=p: (ids[i], p, 0))
        for p in range(_KSPLIT)
    ]
    grid_spec = pltpu.PrefetchScalarGridSpec(
        num_scalar_prefetch=1,
        grid=(n,),
        in_specs=[
            pl.BlockSpec((T, D), lambda i, ids: (0, 0)),
            *we_spec,
            pl.BlockSpec((1, 1, D), lambda i, ids: (ids[i], 0, 0)),
            pl.BlockSpec((1, 1, T), lambda i, ids: (i, 0, 0)),
        ],
        out_specs=pl.BlockSpec((T, D), lambda i, ids: (0, 0)),
    )
    return pl.pallas_call(
        _moe_body,
        grid_spec=grid_spec,
        out_shape=jax.ShapeDtypeStruct((T, D), jnp.float32),
        compiler_params=pltpu.CompilerParams(
            dimension_semantics=("arbitrary",)
        ),
    )(ids, x, *([We] * _KSPLIT), be.reshape(E, 1, D), ws.reshape(E, 1, T))


def kernel(inputs, Wg, bg, We, be, k):
    del k
    gate_logits = _gate(inputs, Wg, bg)
    ids, ws, nact = _route(gate_logits)
    return _moe(ids, nact, inputs, We, be, ws)


# speedup vs baseline: 1.2450x; 1.0733x over previous
import jax
import jax.numpy as jnp
from jax import lax
from jax.experimental import pallas as pl
from jax.experimental.pallas import tpu as pltpu
from jax.experimental.pallas import tpu_sc as plsc

T = 32
E = 64
D = 1024
L = 16
NCH = E // L


def _gate_body(x_ref, wg_ref, bg_ref, o_ref):
    o_ref[...] = (
        jnp.dot(x_ref[...], wg_ref[...], preferred_element_type=jnp.float32)
        + bg_ref[...]
    )


def _gate(x, Wg, bg):
    return pl.pallas_call(
        _gate_body,
        out_shape=jax.ShapeDtypeStruct((T, E), jnp.float32),
    )(x, Wg, bg.reshape(1, E))


def _route_body(logits_hbm, ids_hbm, ws_hbm, nact_hbm, lg, wq, ids_v, ws_v, nact_v):
    c = lax.axis_index("c")
    s = lax.axis_index("s")

    @pl.when(jnp.logical_and(c == 0, s == 0))
    def _():
        pltpu.sync_copy(logits_hbm, lg)
        iota = lax.iota(jnp.int32, L)
        gidx = [iota + L * k for k in range(NCH)]
        NEG = jnp.float32(-1e30)
        BIG = jnp.int32(9999)

        def token_body(t, acc):
            lk = [lg[t, pl.ds(L * k, L)] for k in range(NCH)]
            m = jnp.max(lk[0])
            for k in range(1, NCH):
                m = jnp.maximum(m, jnp.max(lk[k]))
            mv = jnp.full((L,), m, jnp.float32)
            ek = [jnp.exp(lk[k] - mv) for k in range(NCH)]
            Z = jnp.sum(ek[0])
            for k in range(1, NCH):
                Z = Z + jnp.sum(ek[k])
            Zv = jnp.full((L,), Z, jnp.float32)
            idx1 = jnp.min(jnp.where(lk[0] == mv, gidx[0], jnp.full((L,), BIG)))
            for k in range(1, NCH):
                idx1 = jnp.minimum(
                    idx1,
                    jnp.min(jnp.where(lk[k] == mv, gidx[k], jnp.full((L,), BIG))),
                )
            i1v = jnp.full((L,), idx1, jnp.int32)
            l2k = [
                jnp.where(gidx[k] == i1v, jnp.full((L,), NEG), lk[k])
                for k in range(NCH)
            ]
            m2 = jnp.max(l2k[0])
            for k in range(1, NCH):
                m2 = jnp.maximum(m2, jnp.max(l2k[k]))
            m2v = jnp.full((L,), m2, jnp.float32)
            idx2 = jnp.min(jnp.where(l2k[0] == m2v, gidx[0], jnp.full((L,), BIG)))
            for k in range(1, NCH):
                idx2 = jnp.minimum(
                    idx2,
                    jnp.min(jnp.where(l2k[k] == m2v, gidx[k], jnp.full((L,), BIG))),
                )
            i2v = jnp.full((L,), idx2, jnp.int32)
            new_acc = []
            for k in range(NCH):
                selk = jnp.logical_or(gidx[k] == i1v, gidx[k] == i2v)
                wk = jnp.where(selk, ek[k] / Zv, jnp.zeros((L,), jnp.float32))
                wq[t, pl.ds(L * k, L)] = wk
                new_acc.append(jnp.maximum(acc[k], wk))
            return tuple(new_acc)

        zero = jnp.zeros((L,), jnp.float32)
        acc = lax.fori_loop(0, T, token_body, (zero,) * NCH)

        off = jnp.int32(0)
        for k in range(NCH):
            selk = acc[k] > 0.0
            sel_i = jnp.where(selk, jnp.ones((L,), jnp.int32), jnp.zeros((L,), jnp.int32))
            cs = plsc.cumsum(sel_i) + jnp.full((L,), off, jnp.int32)
            pos = jnp.maximum(cs - 1, jnp.zeros((L,), jnp.int32))
            plsc.store_scatter(ids_v, [pos], gidx[k], mask=selk)
            off = off + jnp.sum(sel_i)
        n = off
        lastv = plsc.load_gather(ids_v, [jnp.full((L,), n - 1, jnp.int32)])
        for k in range(NCH):
            cur = ids_v[pl.ds(L * k, L)]
            nv = jnp.full((L,), n, jnp.int32)
            ids_v[pl.ds(L * k, L)] = jnp.where(gidx[k] >= nv, lastv, cur)
        nact_v[...] = jnp.full((L,), n, jnp.int32)

        def ws_body(i, carry):
            ev = plsc.load_gather(ids_v, [jnp.full((L,), i, jnp.int32)])
            scale = jnp.where(i < n, jnp.float32(1.0), jnp.float32(0.0))
            sv = jnp.full((L,), scale, jnp.float32)
            for h in range(T // L):
                tv = iota + L * h
                vals = plsc.load_gather(wq, [tv, ev]) * sv
                ws_v[i, pl.ds(L * h, L)] = vals
            return carry

        lax.fori_loop(0, E, ws_body, jnp.int32(0))

        pltpu.sync_copy(ids_v, ids_hbm)
        pltpu.sync_copy(ws_v, ws_hbm)
        pltpu.sync_copy(nact_v, nact_hbm)


def _route(gate_logits):
    mesh = plsc.VectorSubcoreMesh(core_axis_name="c", subcore_axis_name="s")
    return pl.kernel(
        _route_body,
        compiler_params=pltpu.CompilerParams(needs_layout_passes=False),
        out_type=(
            jax.ShapeDtypeStruct((E,), jnp.int32),
            jax.ShapeDtypeStruct((E, T), jnp.float32),
            jax.ShapeDtypeStruct((L,), jnp.int32),
        ),
        mesh=mesh,
        scratch_types=(
            pltpu.VMEM((T, E), jnp.float32),
            pltpu.VMEM((T, E), jnp.float32),
            pltpu.VMEM((E,), jnp.int32),
            pltpu.VMEM((E, T), jnp.float32),
            pltpu.VMEM((L,), jnp.int32),
        ),
    )(gate_logits)


_KSPLIT = 2
_KS = D // _KSPLIT
_EPG = 2


def _moe_body(ids_ref, x_ref, *rest):
    we_refs = rest[: _EPG * _KSPLIT]
    be_refs = rest[_EPG * _KSPLIT: _EPG * _KSPLIT + _EPG]
    ws_ref, o_ref = rest[_EPG * _KSPLIT + _EPG:]
    i = pl.program_id(0)

    @pl.when(i == 0)
    def _init():
        o_ref[...] = jnp.zeros_like(o_ref)

    upd = None
    for g in range(_EPG):
        y = jnp.dot(
            x_ref[:, pl.ds(0, _KS)], we_refs[g * _KSPLIT][0],
            preferred_element_type=jnp.float32,
        )
        for p in range(1, _KSPLIT):
            y += jnp.dot(
                x_ref[:, pl.ds(p * _KS, _KS)], we_refs[g * _KSPLIT + p][0],
                preferred_element_type=jnp.float32,
            )
        w = ws_ref[0, g, :]
        contrib = (y + be_refs[g][0]) * w[:, None]
        upd = contrib if upd is None else upd + contrib
    o_ref[...] += upd


def _moe(ids, nact, x, We, be, ws):
    n = (nact[0] + (_EPG - 1)) // _EPG
    we_spec = [
        pl.BlockSpec(
            (1, _KS, D),
            lambda i, ids, g=g, p=p: (ids[_EPG * i + g], p, 0),
        )
        for g in range(_EPG)
        for p in range(_KSPLIT)
    ]
    be_spec = [
        pl.BlockSpec((1, 1, D), lambda i, ids, g=g: (ids[_EPG * i + g], 0, 0))
        for g in range(_EPG)
    ]
    grid_spec = pltpu.PrefetchScalarGridSpec(
        num_scalar_prefetch=1,
        grid=(n,),
        in_specs=[
            pl.BlockSpec((T, D), lambda i, ids: (0, 0)),
            *we_spec,
            *be_spec,
            pl.BlockSpec((1, _EPG, T), lambda i, ids: (i, 0, 0)),
        ],
        out_specs=pl.BlockSpec((T, D), lambda i, ids: (0, 0)),
    )
    return pl.pallas_call(
        _moe_body,
        grid_spec=grid_spec,
        out_shape=jax.ShapeDtypeStruct((T, D), jnp.float32),
        compiler_params=pltpu.CompilerParams(
            dimension_semantics=("arbitrary",)
        ),
    )(
        ids, x,
        *([We] * (_EPG * _KSPLIT)),
        *([be.reshape(E, 1, D)] * _EPG),
        ws.reshape(E // _EPG, _EPG, T),
    )


def kernel(inputs, Wg, bg, We, be, k):
    del k
    gate_logits = _gate(inputs, Wg, bg)
    ids, ws, nact = _route(gate_logits)
    return _moe(ids, nact, inputs, We, be, ws)
